# baseline (speedup 1.0000x reference)
import functools

import jax
import jax.numpy as jnp
from jax import lax
from jax.experimental import pallas as pl
from jax.experimental.pallas import tpu as pltpu
from jax.experimental.pallas import tpu_sc as plsc

L = 16
DT = 64
EB = 16
EBG = 64
CHUNK = 1024


def _gather16(vec, idx):
    dnums = lax.GatherDimensionNumbers(
        offset_dims=(), collapsed_slice_dims=(0,), start_index_map=(0,))
    return lax.gather(vec, idx.reshape(L, 1), dnums, slice_sizes=(1,),
                      mode=lax.GatherScatterMode.PROMISE_IN_BOUNDS)


def _bcast_lane(vec, lane):
    return _gather16(vec, jnp.full((L,), lane, jnp.int32))


def _make_sc_spmm(n_vertex, F, NW, TPW):
    NVP = NW * TPW * DT
    mesh = plsc.VectorSubcoreMesh(core_axis_name="c", subcore_axis_name="s")

    @functools.partial(
        pl.kernel,
        out_type=jax.ShapeDtypeStruct((NVP * F,), jnp.float32),
        mesh=mesh,
        compiler_params=pltpu.CompilerParams(
            use_tc_tiling_on_sc=False, needs_layout_passes=False),
        scratch_types=[
            pltpu.VMEM((L,), jnp.int32),
            pltpu.VMEM((CHUNK,), jnp.int32),
            pltpu.VMEM((CHUNK,), jnp.int32),
            pltpu.VMEM((CHUNK,), jnp.float32),
            pltpu.VMEM((EBG, F), jnp.float32),
            pltpu.VMEM((EBG, F), jnp.float32),
            pltpu.VMEM(((DT + 1) * F,), jnp.float32),
            pltpu.SemaphoreType.DMA,
            pltpu.SemaphoreType.DMA,
        ],
    )
    def sc_spmm(s_hbm, src_hbm, dst_hbm, val_hbm, meta_hbm, out_hbm,
                meta_v, src_b, dst_b, val_b, gbuf0, gbuf1, acc, sem0, sem1):
        info = plsc.get_sparse_core_info()
        w = lax.axis_index("c") * info.num_subcores + lax.axis_index("s")
        pltpu.sync_copy(meta_hbm.at[w], meta_v)
        lanes = lax.iota(jnp.int32, L)
        mvec = meta_v[...]

        def tile_body(k, _):
            base = (w * TPW + k) * DT
            es = jnp.sum(jnp.where(lanes == k, mvec, 0))
            ee = jnp.sum(jnp.where(lanes == k + 1, mvec, 0))
            es_al = (es // 8) * 8

            def zero_row(r, _):
                for j in range(F // L):
                    acc[pl.ds(r * F + j * L, L)] = jnp.zeros((L,), jnp.float32)
                return 0

            lax.fori_loop(0, DT, zero_row, 0)

            NJ = F // L

            def flush(regs, cd):
                rowb = cd * F
                for j in range(NJ):
                    plsc.addupdate(acc.at[pl.ds(rowb + j * L, L)], regs[j])

            def chunk_body(ci, carry):
                cs = es_al + ci * CHUNK
                pltpu.sync_copy(src_hbm.at[pl.ds(cs, CHUNK)], src_b)
                pltpu.sync_copy(dst_hbm.at[pl.ds(cs, CHUNK)], dst_b)
                pltpu.sync_copy(val_hbm.at[pl.ds(cs, CHUNK)], val_b)
                cn = jnp.minimum(CHUNK, ee - cs)
                nsb = (cn + EBG - 1) // EBG
                npairs = (nsb + 1) // 2

                def issue(buf, semx, sb):
                    off = jnp.minimum(sb * EBG, CHUNK - EBG)
                    pltpu.async_copy(s_hbm.at[src_b.at[pl.ds(off, EBG)]],
                                     buf, semx)

                def bwait(buf, semx):
                    pltpu.make_async_copy(s_hbm.at[pl.ds(0, EBG)],
                                          buf, semx).wait()

                def process(buf, sb, carry):
                    off = jnp.minimum(sb * EBG, CHUNK - EBG)
                    ok_sb = sb < nsb
                    for s in range(EBG // EB):
                        soff = off + s * EB
                        dstv = dst_b[pl.ds(soff, EB)]
                        valv = val_b[pl.ds(soff, EB)]
                        geid = cs + soff + lanes
                        valid = (geid >= es) & (geid < ee) & ok_sb
                        valm = jnp.where(valid, valv, jnp.zeros_like(valv))
                        dloc = jnp.where(valid,
                                         jnp.clip(dstv - base, 0, DT - 1),
                                         jnp.full((L,), DT, jnp.int32))

                        shifted = _gather16(dloc, jnp.maximum(lanes - 1, 0))
                        prev_v = jnp.full((L,), carry[1], jnp.int32)
                        shifted = jnp.where(lanes == 0, prev_v, shifted)
                        same = jnp.all(dloc == shifted)

                        def fast_fn(ec, s=s, valm=valm):
                            def fast_edge(i, ec2):
                                regs, cur_d = ec2
                                vb = _bcast_lane(valm, i)
                                row = s * EB + i
                                regs = tuple(
                                    regs[j] + vb * buf[row, pl.ds(j * L, L)]
                                    for j in range(NJ))
                                return (regs, cur_d)
                            return lax.fori_loop(0, EB, fast_edge, ec)

                        def slow_fn(ec, s=s, valm=valm, dloc=dloc):
                            def edge_body(i, ec2):
                                regs, cur_d = ec2
                                vb = _bcast_lane(valm, i)
                                d = jnp.sum(jnp.where(lanes == i, dloc, 0))

                                def _flush(ops):
                                    regs_in, cd = ops
                                    flush(regs_in, cd)
                                    return tuple(jnp.zeros((L,), jnp.float32)
                                                 for _ in range(NJ))

                                regs = lax.cond(d != cur_d, _flush,
                                                lambda ops: ops[0],
                                                (regs, cur_d))
                                row = s * EB + i
                                regs = tuple(
                                    regs[j] + vb * buf[row, pl.ds(j * L, L)]
                                    for j in range(NJ))
                                return (regs, d)
                            return lax.fori_loop(0, EB, edge_body, ec)

                        carry = lax.cond(same, fast_fn, slow_fn, carry)
                    return carry

                @pl.when(npairs > 0)
                def _():
                    issue(gbuf0, sem0, 0)

                def pair_body(p, carry):
                    b0 = 2 * p
                    issue(gbuf1, sem1, b0 + 1)
                    bwait(gbuf0, sem0)
                    carry = process(gbuf0, b0, carry)
                    issue(gbuf0, sem0, b0 + 2)
                    bwait(gbuf1, sem1)
                    carry = process(gbuf1, b0 + 1, carry)
                    return carry

                carry = lax.fori_loop(0, npairs, pair_body, carry)

                @pl.when(npairs > 0)
                def _():
                    bwait(gbuf0, sem0)

                return carry

            nchunks = (ee - es_al + CHUNK - 1) // CHUNK
            carry0 = (tuple(jnp.zeros((L,), jnp.float32) for _ in range(NJ)),
                      jnp.int32(DT))
            regs, cur_d = lax.fori_loop(0, nchunks, chunk_body, carry0)
            flush(regs, cur_d)
            pltpu.sync_copy(acc.at[pl.ds(0, DT * F)],
                            out_hbm.at[pl.ds(base * F, DT * F)])
            return 0

        lax.fori_loop(0, TPW, tile_body, 0)

    return sc_spmm


def _mm_body(x_ref, w_ref, b_ref, o_ref):
    o_ref[...] = (
        jnp.dot(x_ref[...], w_ref[...], preferred_element_type=jnp.float32)
        + b_ref[...]
    )


def kernel(x, weight, bias, filter_vals, edge_src, edge_dst):
    Bsz, c_in, Tlen, n_vertex = x.shape
    c_out = weight.shape[1]
    G = Bsz * Tlen
    F = G * c_in
    E = edge_src.shape[0]

    NW = 32
    NT = -(-n_vertex // DT)
    TPW = -(-NT // NW)

    S = x.reshape(n_vertex, F)
    src = edge_src.astype(jnp.int32)
    dst = edge_dst.astype(jnp.int32)
    vals = filter_vals.astype(jnp.float32)

    ts = jnp.searchsorted(
        dst, jnp.arange(NW * TPW + 1, dtype=jnp.int32) * DT
    ).astype(jnp.int32)
    meta = jnp.zeros((NW, L), jnp.int32)
    cols = jnp.arange(NW)[:, None] * TPW + jnp.arange(TPW + 1)[None, :]
    meta = meta.at[:, : TPW + 1].set(ts[cols])

    pad = CHUNK + 8
    src_p = jnp.concatenate([src, jnp.zeros((pad,), jnp.int32)])
    dst_p = jnp.concatenate([dst, jnp.zeros((pad,), jnp.int32)])
    val_p = jnp.concatenate([vals, jnp.zeros((pad,), jnp.float32)])

    q_pad = _make_sc_spmm(n_vertex, F, NW, TPW)(S, src_p, dst_p, val_p, meta)
    Q = q_pad[: n_vertex * F].reshape(n_vertex * G, c_in)

    R = n_vertex * G
    BLK = 1600
    out = pl.pallas_call(
        _mm_body,
        grid=(R // BLK,),
        in_specs=[
            pl.BlockSpec((BLK, c_in), lambda i: (i, 0)),
            pl.BlockSpec((c_in, c_out), lambda i: (0, 0)),
            pl.BlockSpec((1, c_out), lambda i: (0, 0)),
        ],
        out_specs=pl.BlockSpec((BLK, c_out), lambda i: (i, 0)),
        out_shape=jax.ShapeDtypeStruct((R, c_out), jnp.float32),
    )(Q, weight, bias.reshape(1, c_out))
    return out


# speedup vs baseline: 1.2904x; 1.2904x over previous
import functools

import jax
import jax.numpy as jnp
from jax import lax
from jax.experimental import pallas as pl
from jax.experimental.pallas import tpu as pltpu
from jax.experimental.pallas import tpu_sc as plsc

L = 16
DT = 64
EB = 16
EBG = 64
CHUNK = 2048


def _gather16(vec, idx):
    dnums = lax.GatherDimensionNumbers(
        offset_dims=(), collapsed_slice_dims=(0,), start_index_map=(0,))
    return lax.gather(vec, idx.reshape(L, 1), dnums, slice_sizes=(1,),
                      mode=lax.GatherScatterMode.PROMISE_IN_BOUNDS)


def _bcast_lane(vec, lane):
    return _gather16(vec, jnp.full((L,), lane, jnp.int32))


def _make_sc_spmm(n_vertex, F, NW, TPW):
    NVP = NW * TPW * DT
    mesh = plsc.VectorSubcoreMesh(core_axis_name="c", subcore_axis_name="s")

    @functools.partial(
        pl.kernel,
        out_type=jax.ShapeDtypeStruct((NVP * F,), jnp.float32),
        mesh=mesh,
        compiler_params=pltpu.CompilerParams(
            use_tc_tiling_on_sc=False, needs_layout_passes=False),
        scratch_types=[
            pltpu.VMEM((L,), jnp.int32),
            pltpu.VMEM((CHUNK,), jnp.int32),
            pltpu.VMEM((CHUNK,), jnp.int32),
            pltpu.VMEM((CHUNK,), jnp.float32),
            pltpu.VMEM((EBG, F), jnp.float32),
            pltpu.VMEM((EBG, F), jnp.float32),
            pltpu.VMEM(((DT + 1) * F,), jnp.float32),
            pltpu.SemaphoreType.DMA,
            pltpu.SemaphoreType.DMA,
        ],
    )
    def sc_spmm(s_hbm, src_hbm, dst_hbm, val_hbm, meta_hbm, out_hbm,
                meta_v, src_b, dst_b, val_b, gbuf0, gbuf1, acc, sem0, sem1):
        info = plsc.get_sparse_core_info()
        w = lax.axis_index("c") * info.num_subcores + lax.axis_index("s")
        pltpu.sync_copy(meta_hbm.at[w], meta_v)
        lanes = lax.iota(jnp.int32, L)
        mvec = meta_v[...]

        def tile_body(k, _):
            base = (w * TPW + k) * DT
            es = jnp.sum(jnp.where(lanes == k, mvec, 0))
            ee = jnp.sum(jnp.where(lanes == k + 1, mvec, 0))
            es_al = (es // 8) * 8

            def zero_row(r, _):
                for j in range(F // L):
                    acc[pl.ds(r * F + j * L, L)] = jnp.zeros((L,), jnp.float32)
                return 0

            lax.fori_loop(0, DT, zero_row, 0)

            NJ = F // L

            def flush(regs, cd):
                rowb = cd * F
                for j in range(NJ):
                    plsc.addupdate(acc.at[pl.ds(rowb + j * L, L)], regs[j])

            def chunk_body(ci, carry):
                cs = es_al + ci * CHUNK
                c1 = pltpu.async_copy(src_hbm.at[pl.ds(cs, CHUNK)], src_b,
                                      sem0)
                c2 = pltpu.async_copy(dst_hbm.at[pl.ds(cs, CHUNK)], dst_b,
                                      sem0)
                c3 = pltpu.async_copy(val_hbm.at[pl.ds(cs, CHUNK)], val_b,
                                      sem0)
                c1.wait()
                c2.wait()
                c3.wait()
                cn = jnp.minimum(CHUNK, ee - cs)
                nsb = (cn + EBG - 1) // EBG
                npairs = (nsb + 1) // 2

                def issue(buf, semx, sb):
                    off = jnp.minimum(sb * EBG, CHUNK - EBG)
                    pltpu.async_copy(s_hbm.at[src_b.at[pl.ds(off, EBG)]],
                                     buf, semx)

                def bwait(buf, semx):
                    pltpu.make_async_copy(s_hbm.at[pl.ds(0, EBG)],
                                          buf, semx).wait()

                def process(buf, sb, carry):
                    off = jnp.minimum(sb * EBG, CHUNK - EBG)
                    ok_sb = sb < nsb
                    for s in range(EBG // EB):
                        soff = off + s * EB
                        dstv = dst_b[pl.ds(soff, EB)]
                        valv = val_b[pl.ds(soff, EB)]
                        geid = cs + soff + lanes
                        valid = (geid >= es) & (geid < ee) & ok_sb
                        valm = jnp.where(valid, valv, jnp.zeros_like(valv))
                        dloc = jnp.where(valid,
                                         jnp.clip(dstv - base, 0, DT - 1),
                                         jnp.full((L,), DT, jnp.int32))

                        def slow_fn(ec, s=s, valm=valm, dloc=dloc):
                            def edge_body(i, ec2):
                                regs, cur_d = ec2
                                vb = _bcast_lane(valm, i)
                                d = jnp.sum(jnp.where(lanes == i, dloc, 0))

                                def _flush(ops):
                                    regs_in, cd = ops
                                    flush(regs_in, cd)
                                    return tuple(jnp.zeros((L,), jnp.float32)
                                                 for _ in range(NJ))

                                regs = lax.cond(d != cur_d, _flush,
                                                lambda ops: ops[0],
                                                (regs, cur_d))
                                row = s * EB + i
                                regs = tuple(
                                    regs[j] + vb * buf[row, pl.ds(j * L, L)]
                                    for j in range(NJ))
                                return (regs, d)
                            return lax.fori_loop(0, EB, edge_body, ec)

                        carry = slow_fn(carry)
                    return carry

                @pl.when(npairs > 0)
                def _():
                    issue(gbuf0, sem0, 0)

                def pair_body(p, carry):
                    b0 = 2 * p
                    issue(gbuf1, sem1, b0 + 1)
                    bwait(gbuf0, sem0)
                    carry = process(gbuf0, b0, carry)
                    issue(gbuf0, sem0, b0 + 2)
                    bwait(gbuf1, sem1)
                    carry = process(gbuf1, b0 + 1, carry)
                    return carry

                carry = lax.fori_loop(0, npairs, pair_body, carry)

                @pl.when(npairs > 0)
                def _():
                    bwait(gbuf0, sem0)

                return carry

            nchunks = (ee - es_al + CHUNK - 1) // CHUNK
            carry0 = (tuple(jnp.zeros((L,), jnp.float32) for _ in range(NJ)),
                      jnp.int32(DT))
            regs, cur_d = lax.fori_loop(0, nchunks, chunk_body, carry0)
            flush(regs, cur_d)
            pltpu.sync_copy(acc.at[pl.ds(0, DT * F)],
                            out_hbm.at[pl.ds(base * F, DT * F)])
            return 0

        lax.fori_loop(0, TPW, tile_body, 0)

    return sc_spmm


def _mm_body(x_ref, w_ref, b_ref, o_ref):
    o_ref[...] = (
        jnp.dot(x_ref[...], w_ref[...], preferred_element_type=jnp.float32)
        + b_ref[...]
    )


def kernel(x, weight, bias, filter_vals, edge_src, edge_dst):
    Bsz, c_in, Tlen, n_vertex = x.shape
    c_out = weight.shape[1]
    G = Bsz * Tlen
    F = G * c_in

    NW = 32
    NT = -(-n_vertex // DT)
    TPW = -(-NT // NW)

    S = x.reshape(n_vertex, F)
    src = edge_src.astype(jnp.int32)
    dst = edge_dst.astype(jnp.int32)
    vals = filter_vals.astype(jnp.float32)

    ts = jnp.searchsorted(
        dst, jnp.arange(NW * TPW + 1, dtype=jnp.int32) * DT
    ).astype(jnp.int32)
    meta = jnp.zeros((NW, L), jnp.int32)
    cols = jnp.arange(NW)[:, None] * TPW + jnp.arange(TPW + 1)[None, :]
    meta = meta.at[:, : TPW + 1].set(ts[cols])

    pad = CHUNK + 8
    src_p = jnp.concatenate([src, jnp.zeros((pad,), jnp.int32)])
    dst_p = jnp.concatenate([dst, jnp.zeros((pad,), jnp.int32)])
    val_p = jnp.concatenate([vals, jnp.zeros((pad,), jnp.float32)])

    q_pad = _make_sc_spmm(n_vertex, F, NW, TPW)(S, src_p, dst_p, val_p, meta)
    Q = q_pad.reshape(-1, c_in)

    R = n_vertex * G
    BLK = 1600
    out = pl.pallas_call(
        _mm_body,
        grid=(R // BLK,),
        in_specs=[
            pl.BlockSpec((BLK, c_in), lambda i: (i, 0)),
            pl.BlockSpec((c_in, c_out), lambda i: (0, 0)),
            pl.BlockSpec((1, c_out), lambda i: (0, 0)),
        ],
        out_specs=pl.BlockSpec((BLK, c_out), lambda i: (i, 0)),
        out_shape=jax.ShapeDtypeStruct((R, c_out), jnp.float32),
    )(Q, weight, bias.reshape(1, c_out))
    return out
